# Initial kernel scaffold; baseline (speedup 1.0000x reference)
import jax
import jax.numpy as jnp
from jax.experimental import pallas as pl


def kernel(msg, index, t, dim_size):
    raise NotImplementedError("write your pallas kernel here")


# speedup vs baseline: 9.9662x; 9.9662x over previous
import functools

import jax
import jax.numpy as jnp
from jax import lax
from jax.experimental import pallas as pl
from jax.experimental.pallas import tpu as pltpu
from jax.experimental.pallas import tpu_sc as plsc

N = 160000
D = 256
DIM = 10000
DIMP = 10240
NC = 2
NS = 16
L = 16
EPT = N // NS
CHUNKS = EPT // L
SLICE = DIMP // NS
OUT_PER = DIMP // (NC * NS)
GCH = 80
NGCH = OUT_PER // GCH

_NEG_INF = float(jnp.finfo(jnp.float32).min)


def _rmw_max(arr_ref, idx, val, zero):
    cur = plsc.load_gather(arr_ref, [idx])
    better = val > cur
    plsc.store_scatter(arr_ref, [idx], val, mask=better)
    chk = plsc.load_gather(arr_ref, [idx])
    retry = jnp.logical_and(better, chk < val).astype(jnp.int32)

    def cond(m):
        return jnp.max(m) > 0

    def body(m):
        act = m > 0
        cur2 = plsc.load_gather(arr_ref, [idx])
        b = jnp.logical_and(act, val > cur2)
        plsc.store_scatter(arr_ref, [idx], val, mask=b)
        chk2 = plsc.load_gather(arr_ref, [idx])
        return jnp.logical_and(b, chk2 < val).astype(jnp.int32)

    lax.while_loop(cond, body, retry)
    del zero


def _body(msg_hbm, idx_hbm, t_hbm, ninf_hbm, neg1_hbm, out_hbm,
          idx_v, t_v, tseg_v, gseg_v, pmax_v, red_v, gs_v,
          red_p, gs_p, amax_v, safe_v, rows_v, sem,
          sp_t, sp_g, sp_p, sp_a):
    c = lax.axis_index("c")
    s = lax.axis_index("s")
    base_in = s * EPT

    pltpu.sync_copy(idx_hbm.at[pl.ds(base_in, EPT)], idx_v)
    pltpu.sync_copy(t_hbm.at[pl.ds(base_in, EPT)], t_v)
    pltpu.sync_copy(ninf_hbm, tseg_v)
    pltpu.sync_copy(neg1_hbm, pmax_v)

    def p1(i, _):
        idx = idx_v[pl.ds(i * L, L)]
        tv = t_v[pl.ds(i * L, L)]
        _rmw_max(tseg_v, idx, tv, 0.0)
        return _

    lax.fori_loop(0, CHUNKS, p1, None)

    pltpu.sync_copy(tseg_v, sp_t.at[s])
    plsc.subcore_barrier()
    pltpu.sync_copy(sp_t.at[:, pl.ds(s * SLICE, SLICE)], red_v)

    def red_f32(j, _):
        acc = red_v[0, pl.ds(j * L, L)]
        for k in range(1, NS):
            acc = jnp.maximum(acc, red_v[k, pl.ds(j * L, L)])
        gs_v[pl.ds(j * L, L)] = acc
        return _

    lax.fori_loop(0, SLICE // L, red_f32, None)
    pltpu.sync_copy(gs_v, sp_g.at[pl.ds(s * SLICE, SLICE)])
    plsc.subcore_barrier()
    pltpu.sync_copy(sp_g, gseg_v)

    def p2(i, _):
        idx = idx_v[pl.ds(i * L, L)]
        tv = t_v[pl.ds(i * L, L)]
        g = plsc.load_gather(gseg_v, [idx])
        pos = base_in + i * L + lax.iota(jnp.int32, L)
        cand = jnp.where(tv == g, pos, jnp.int32(-1))
        _rmw_max(pmax_v, idx, cand, 0)
        return _

    lax.fori_loop(0, CHUNKS, p2, None)

    pltpu.sync_copy(pmax_v, sp_p.at[s])
    plsc.subcore_barrier()
    pltpu.sync_copy(sp_p.at[:, pl.ds(s * SLICE, SLICE)], red_p)

    def red_i32(j, _):
        acc = red_p[0, pl.ds(j * L, L)]
        for k in range(1, NS):
            acc = jnp.maximum(acc, red_p[k, pl.ds(j * L, L)])
        gs_p[pl.ds(j * L, L)] = acc
        return _

    lax.fori_loop(0, SLICE // L, red_i32, None)
    pltpu.sync_copy(gs_p, sp_a.at[pl.ds(s * SLICE, SLICE)])
    plsc.subcore_barrier()

    obase = c * (NS * OUT_PER) + s * OUT_PER
    pltpu.sync_copy(sp_a.at[pl.ds(obase, OUT_PER)], amax_v)
    for j in range(OUT_PER // L):
        a = amax_v[pl.ds(j * L, L)]
        safe_v[j // (GCH // L), pl.ds((j % (GCH // L)) * L, L)] = (
            jnp.maximum(a, 0))

    lane = lax.iota(jnp.int32, L)
    for cc in range(NGCH):
        pltpu.async_copy(msg_hbm.at[safe_v.at[cc]], rows_v, sem).wait()

        def fix(g, _):
            a16 = amax_v[pl.ds(cc * GCH + g * L, L)]
            any_invalid = jnp.min(a16) < 0

            @pl.when(any_invalid)
            def _zero():
                for r in range(L):
                    a_r = jnp.sum(jnp.where(lane == r, a16, 0))
                    m = jnp.where(a_r < 0, jnp.float32(0), jnp.float32(1))
                    row = g * L + r
                    for k in range(D // L):
                        rows_v[row, pl.ds(k * L, L)] = (
                            rows_v[row, pl.ds(k * L, L)] * m)

            return _

        lax.fori_loop(0, GCH // L, fix, None)
        pltpu.sync_copy(rows_v, out_hbm.at[pl.ds(obase + cc * GCH, GCH)])


@functools.partial(jax.jit, static_argnums=())
def kernel(msg, index, t, dim_size):
    del dim_size
    ninf = jnp.full((DIMP,), _NEG_INF, dtype=jnp.float32)
    neg1 = jnp.full((DIMP,), -1, dtype=jnp.int32)

    mesh = plsc.VectorSubcoreMesh(
        core_axis_name="c", subcore_axis_name="s",
        num_cores=NC, num_subcores=NS)
    run = pl.kernel(
        _body,
        out_type=jax.ShapeDtypeStruct((DIMP, D), jnp.float32),
        mesh=mesh,
        compiler_params=pltpu.CompilerParams(needs_layout_passes=False),
        scratch_types=[
            pltpu.VMEM((EPT,), jnp.int32),
            pltpu.VMEM((EPT,), jnp.float32),
            pltpu.VMEM((DIMP,), jnp.float32),
            pltpu.VMEM((DIMP,), jnp.float32),
            pltpu.VMEM((DIMP,), jnp.int32),
            pltpu.VMEM((NS, SLICE), jnp.float32),
            pltpu.VMEM((SLICE,), jnp.float32),
            pltpu.VMEM((NS, SLICE), jnp.int32),
            pltpu.VMEM((SLICE,), jnp.int32),
            pltpu.VMEM((OUT_PER,), jnp.int32),
            pltpu.VMEM((NGCH, GCH), jnp.int32),
            pltpu.VMEM((GCH, D), jnp.float32),
            pltpu.SemaphoreType.DMA,
            pltpu.VMEM_SHARED((NS, DIMP), jnp.float32),
            pltpu.VMEM_SHARED((DIMP,), jnp.float32),
            pltpu.VMEM_SHARED((NS, DIMP), jnp.int32),
            pltpu.VMEM_SHARED((DIMP,), jnp.int32),
        ],
    )
    out = run(msg, index, t, ninf, neg1)
    return out[:DIM]


# speedup vs baseline: 13.1244x; 1.3169x over previous
import functools

import jax
import jax.numpy as jnp
from jax import lax
from jax.experimental import pallas as pl
from jax.experimental.pallas import tpu as pltpu
from jax.experimental.pallas import tpu_sc as plsc

N = 160000
D = 256
DIM = 10000
DIMP = 10240
NC = 2
NS = 16
L = 16
EPT = N // NS
CHUNKS = EPT // L
SLICE = DIMP // NS
OUT_PER = DIMP // (NC * NS)
GCH = 80
NGCH = OUT_PER // GCH

_NEG_INF = float(jnp.finfo(jnp.float32).min)


def _any_lane(mask):
    cnt = plsc.all_reduce_population_count(mask)
    return jnp.squeeze(lax.slice(cnt, (0,), (1,))) > 0


def _rmw_max(arr_ref, idx, val):
    cur = plsc.load_gather(arr_ref, [idx])
    better = val > cur
    plsc.store_scatter(arr_ref, [idx], val, mask=better)
    chk = plsc.load_gather(arr_ref, [idx])
    retry = jnp.logical_and(better, chk < val).astype(jnp.int32)

    def cond(m):
        return _any_lane(m > 0)

    def body(m):
        act = m > 0
        cur2 = plsc.load_gather(arr_ref, [idx])
        b = jnp.logical_and(act, val > cur2)
        plsc.store_scatter(arr_ref, [idx], val, mask=b)
        chk2 = plsc.load_gather(arr_ref, [idx])
        return jnp.logical_and(b, chk2 < val).astype(jnp.int32)

    lax.while_loop(cond, body, retry)


def _body(msg_hbm, idx_hbm, t_hbm, ninf_hbm, neg1_hbm, out_hbm,
          idx_v, t_v, tseg_v, gseg_v, pmax_v, red_v, gs_v,
          red_p, gs_p, amax_v, safe_v, rows_v, sem,
          sp_t, sp_g, sp_p, sp_a):
    c = lax.axis_index("c")
    s = lax.axis_index("s")
    base_in = s * EPT

    pltpu.sync_copy(idx_hbm.at[pl.ds(base_in, EPT)], idx_v)
    pltpu.sync_copy(t_hbm.at[pl.ds(base_in, EPT)], t_v)
    pltpu.sync_copy(ninf_hbm, tseg_v)
    pltpu.sync_copy(neg1_hbm, pmax_v)

    def p1(i, _):
        idx = idx_v[pl.ds(i * L, L)]
        tv = t_v[pl.ds(i * L, L)]
        _rmw_max(tseg_v, idx, tv)
        return _

    lax.fori_loop(0, CHUNKS, p1, None)

    pltpu.sync_copy(tseg_v, sp_t.at[s])
    plsc.subcore_barrier()
    pltpu.sync_copy(sp_t.at[:, pl.ds(s * SLICE, SLICE)], red_v)

    def red_f32(j, _):
        acc = red_v[0, pl.ds(j * L, L)]
        for k in range(1, NS):
            acc = jnp.maximum(acc, red_v[k, pl.ds(j * L, L)])
        gs_v[pl.ds(j * L, L)] = acc
        return _

    lax.fori_loop(0, SLICE // L, red_f32, None)
    pltpu.sync_copy(gs_v, sp_g.at[pl.ds(s * SLICE, SLICE)])
    plsc.subcore_barrier()
    pltpu.sync_copy(sp_g, gseg_v)

    def p2(i, _):
        idx = idx_v[pl.ds(i * L, L)]
        tv = t_v[pl.ds(i * L, L)]
        g = plsc.load_gather(gseg_v, [idx])
        pos = base_in + i * L + lax.iota(jnp.int32, L)
        cand = jnp.where(tv == g, pos, jnp.int32(-1))
        _rmw_max(pmax_v, idx, cand)
        return _

    lax.fori_loop(0, CHUNKS, p2, None)

    pltpu.sync_copy(pmax_v, sp_p.at[s])
    plsc.subcore_barrier()
    pltpu.sync_copy(sp_p.at[:, pl.ds(s * SLICE, SLICE)], red_p)

    def red_i32(j, _):
        acc = red_p[0, pl.ds(j * L, L)]
        for k in range(1, NS):
            acc = jnp.maximum(acc, red_p[k, pl.ds(j * L, L)])
        gs_p[pl.ds(j * L, L)] = acc
        return _

    lax.fori_loop(0, SLICE // L, red_i32, None)
    pltpu.sync_copy(gs_p, sp_a.at[pl.ds(s * SLICE, SLICE)])
    plsc.subcore_barrier()

    obase = c * (NS * OUT_PER) + s * OUT_PER
    pltpu.sync_copy(sp_a.at[pl.ds(obase, OUT_PER)], amax_v)
    for j in range(OUT_PER // L):
        a = amax_v[pl.ds(j * L, L)]
        safe_v[j // (GCH // L), pl.ds((j % (GCH // L)) * L, L)] = (
            jnp.maximum(a, 0))

    lane = lax.iota(jnp.int32, L)
    for cc in range(NGCH):
        @pl.when(obase + cc * GCH + GCH <= DIM)
        def _chunk():
            pltpu.async_copy(msg_hbm.at[safe_v.at[cc]], rows_v, sem).wait()

            def fix(g, _):
                a16 = amax_v[pl.ds(cc * GCH + g * L, L)]
                any_invalid = jnp.min(a16) < 0

                @pl.when(any_invalid)
                def _zero():
                    for r in range(L):
                        a_r = jnp.sum(jnp.where(lane == r, a16, 0))
                        m = jnp.where(a_r < 0, jnp.float32(0), jnp.float32(1))
                        row = g * L + r
                        for k in range(D // L):
                            rows_v[row, pl.ds(k * L, L)] = (
                                rows_v[row, pl.ds(k * L, L)] * m)

                return _

            lax.fori_loop(0, GCH // L, fix, None)
            pltpu.sync_copy(rows_v, out_hbm.at[pl.ds(obase + cc * GCH, GCH)])


@functools.partial(jax.jit, static_argnums=())
def kernel(msg, index, t, dim_size):
    del dim_size
    ninf = jnp.full((DIMP,), _NEG_INF, dtype=jnp.float32)
    neg1 = jnp.full((DIMP,), -1, dtype=jnp.int32)

    mesh = plsc.VectorSubcoreMesh(
        core_axis_name="c", subcore_axis_name="s",
        num_cores=NC, num_subcores=NS)
    run = pl.kernel(
        _body,
        out_type=jax.ShapeDtypeStruct((DIM, D), jnp.float32),
        mesh=mesh,
        compiler_params=pltpu.CompilerParams(needs_layout_passes=False),
        scratch_types=[
            pltpu.VMEM((EPT,), jnp.int32),
            pltpu.VMEM((EPT,), jnp.float32),
            pltpu.VMEM((DIMP,), jnp.float32),
            pltpu.VMEM((DIMP,), jnp.float32),
            pltpu.VMEM((DIMP,), jnp.int32),
            pltpu.VMEM((NS, SLICE), jnp.float32),
            pltpu.VMEM((SLICE,), jnp.float32),
            pltpu.VMEM((NS, SLICE), jnp.int32),
            pltpu.VMEM((SLICE,), jnp.int32),
            pltpu.VMEM((OUT_PER,), jnp.int32),
            pltpu.VMEM((NGCH, GCH), jnp.int32),
            pltpu.VMEM((GCH, D), jnp.float32),
            pltpu.SemaphoreType.DMA,
            pltpu.VMEM_SHARED((NS, DIMP), jnp.float32),
            pltpu.VMEM_SHARED((DIMP,), jnp.float32),
            pltpu.VMEM_SHARED((NS, DIMP), jnp.int32),
            pltpu.VMEM_SHARED((DIMP,), jnp.int32),
        ],
    )
    return run(msg, index, t, ninf, neg1)


# speedup vs baseline: 17.8432x; 1.3595x over previous
import functools

import jax
import jax.numpy as jnp
from jax import lax
from jax.experimental import pallas as pl
from jax.experimental.pallas import tpu as pltpu
from jax.experimental.pallas import tpu_sc as plsc

N = 160000
D = 256
DIM = 10000
DIMP = 10240
NC = 2
NS = 16
L = 16
EPT = N // NS
CHUNKS = EPT // L
SLICE = DIMP // NS
OUT_PER = DIMP // (NC * NS)
GCH = 80
NGCH = OUT_PER // GCH

_NEG_INF = float(jnp.finfo(jnp.float32).min)


def _any_lane(mask):
    cnt = plsc.all_reduce_population_count(mask)
    return jnp.squeeze(lax.slice(cnt, (0,), (1,))) > 0


def _lex_update(tseg_v, pmax_v, tmp_v, idx, tv, pos, act):
    plsc.store_scatter(tmp_v, [idx], pos, mask=act)
    w = plsc.load_gather(tmp_v, [idx])
    win = jnp.logical_and(act, w == pos)
    ct = plsc.load_gather(tseg_v, [idx])
    cp = plsc.load_gather(pmax_v, [idx])
    bet = jnp.logical_or(tv > ct,
                         jnp.logical_and(tv == ct, pos > cp))
    wr = jnp.logical_and(win, bet)
    plsc.store_scatter(tseg_v, [idx], tv, mask=wr)
    plsc.store_scatter(pmax_v, [idx], pos, mask=wr)
    return jnp.logical_and(jnp.logical_and(act, jnp.logical_not(win)), bet)


def _body(msg_hbm, idx_hbm, t_hbm, ninf_hbm, neg1_hbm, out_hbm,
          idx_v, t_v, tseg_v, pmax_v, tmp_v, red_v,
          red_p, gs_p, amax_v, safe_v, rows_v, sem,
          sp_t, sp_p, sp_a):
    c = lax.axis_index("c")
    s = lax.axis_index("s")
    base_in = s * EPT
    lane = lax.iota(jnp.int32, L)

    pltpu.sync_copy(idx_hbm.at[pl.ds(base_in, EPT)], idx_v)
    pltpu.sync_copy(t_hbm.at[pl.ds(base_in, EPT)], t_v)
    pltpu.sync_copy(ninf_hbm, tseg_v)
    pltpu.sync_copy(neg1_hbm, pmax_v)
    pltpu.sync_copy(neg1_hbm, tmp_v)

    def walk(i, _):
        idx = idx_v[pl.ds(i * L, L)]
        tv = t_v[pl.ds(i * L, L)]
        pos = base_in + i * L + lane
        all_act = lane >= 0
        rem = _lex_update(tseg_v, pmax_v, tmp_v, idx, tv, pos, all_act)

        @pl.when(_any_lane(rem))
        def _slow():
            def cond(m):
                return _any_lane(m > 0)

            def body(m):
                return _lex_update(
                    tseg_v, pmax_v, tmp_v, idx, tv, pos, m > 0
                ).astype(jnp.int32)

            lax.while_loop(cond, body, rem.astype(jnp.int32))

        return _

    lax.fori_loop(0, CHUNKS, walk, None)

    pltpu.sync_copy(tseg_v, sp_t.at[s])
    pltpu.sync_copy(pmax_v, sp_p.at[s])
    plsc.subcore_barrier()
    pltpu.sync_copy(sp_t.at[:, pl.ds(s * SLICE, SLICE)], red_v)
    pltpu.sync_copy(sp_p.at[:, pl.ds(s * SLICE, SLICE)], red_p)

    def red(j, _):
        ta = red_v[0, pl.ds(j * L, L)]
        pa = red_p[0, pl.ds(j * L, L)]
        for k in range(1, NS):
            tk = red_v[k, pl.ds(j * L, L)]
            pk = red_p[k, pl.ds(j * L, L)]
            b = jnp.logical_or(tk > ta,
                               jnp.logical_and(tk == ta, pk > pa))
            ta = jnp.where(b, tk, ta)
            pa = jnp.where(b, pk, pa)
        gs_p[pl.ds(j * L, L)] = pa
        return _

    lax.fori_loop(0, SLICE // L, red, None)
    pltpu.sync_copy(gs_p, sp_a.at[pl.ds(s * SLICE, SLICE)])
    plsc.subcore_barrier()

    obase = c * (NS * OUT_PER) + s * OUT_PER
    pltpu.sync_copy(sp_a.at[pl.ds(obase, OUT_PER)], amax_v)
    for j in range(OUT_PER // L):
        a = amax_v[pl.ds(j * L, L)]
        safe_v[j // (GCH // L), pl.ds((j % (GCH // L)) * L, L)] = (
            jnp.maximum(a, 0))

    for cc in range(NGCH):
        @pl.when(obase + cc * GCH + GCH <= DIM)
        def _chunk():
            pltpu.async_copy(msg_hbm.at[safe_v.at[cc]], rows_v, sem).wait()

            def fix(g, _):
                a16 = amax_v[pl.ds(cc * GCH + g * L, L)]
                any_invalid = jnp.min(a16) < 0

                @pl.when(any_invalid)
                def _zero():
                    for r in range(L):
                        a_r = jnp.sum(jnp.where(lane == r, a16, 0))
                        m = jnp.where(a_r < 0, jnp.float32(0), jnp.float32(1))
                        row = g * L + r
                        for k in range(D // L):
                            rows_v[row, pl.ds(k * L, L)] = (
                                rows_v[row, pl.ds(k * L, L)] * m)

                return _

            lax.fori_loop(0, GCH // L, fix, None)
            pltpu.sync_copy(rows_v, out_hbm.at[pl.ds(obase + cc * GCH, GCH)])


@functools.partial(jax.jit, static_argnums=())
def kernel(msg, index, t, dim_size):
    del dim_size
    ninf = jnp.full((DIMP,), _NEG_INF, dtype=jnp.float32)
    neg1 = jnp.full((DIMP,), -1, dtype=jnp.int32)

    mesh = plsc.VectorSubcoreMesh(
        core_axis_name="c", subcore_axis_name="s",
        num_cores=NC, num_subcores=NS)
    run = pl.kernel(
        _body,
        out_type=jax.ShapeDtypeStruct((DIM, D), jnp.float32),
        mesh=mesh,
        compiler_params=pltpu.CompilerParams(needs_layout_passes=False),
        scratch_types=[
            pltpu.VMEM((EPT,), jnp.int32),
            pltpu.VMEM((EPT,), jnp.float32),
            pltpu.VMEM((DIMP,), jnp.float32),
            pltpu.VMEM((DIMP,), jnp.int32),
            pltpu.VMEM((DIMP,), jnp.int32),
            pltpu.VMEM((NS, SLICE), jnp.float32),
            pltpu.VMEM((NS, SLICE), jnp.int32),
            pltpu.VMEM((SLICE,), jnp.int32),
            pltpu.VMEM((OUT_PER,), jnp.int32),
            pltpu.VMEM((NGCH, GCH), jnp.int32),
            pltpu.VMEM((GCH, D), jnp.float32),
            pltpu.SemaphoreType.DMA,
            pltpu.VMEM_SHARED((NS, DIMP), jnp.float32),
            pltpu.VMEM_SHARED((NS, DIMP), jnp.int32),
            pltpu.VMEM_SHARED((DIMP,), jnp.int32),
        ],
    )
    return run(msg, index, t, ninf, neg1)


# speedup vs baseline: 19.0890x; 1.0698x over previous
import functools

import jax
import jax.numpy as jnp
from jax import lax
from jax.experimental import pallas as pl
from jax.experimental.pallas import tpu as pltpu
from jax.experimental.pallas import tpu_sc as plsc

N = 160000
D = 256
DIM = 10000
DIMP = 10240
NC = 2
NS = 16
L = 16
EPT = N // NS
CHUNKS = EPT // L
SLICE = DIMP // NS
OUT_PER = DIMP // (NC * NS)
GCH = 80
NGCH = OUT_PER // GCH
U = 4

_NEG_INF = float(jnp.finfo(jnp.float32).min)


def _any_lane(mask):
    cnt = plsc.all_reduce_population_count(mask)
    return jnp.squeeze(lax.slice(cnt, (0,), (1,))) > 0


def _lex_update(tseg_v, pmax_v, tmp_v, idx, tv, pos, act):
    plsc.store_scatter(tmp_v, [idx], pos, mask=act)
    w = plsc.load_gather(tmp_v, [idx])
    win = jnp.logical_and(act, w == pos)
    ct = plsc.load_gather(tseg_v, [idx])
    cp = plsc.load_gather(pmax_v, [idx])
    bet = jnp.logical_or(tv > ct,
                         jnp.logical_and(tv == ct, pos > cp))
    wr = jnp.logical_and(win, bet)
    plsc.store_scatter(tseg_v, [idx], tv, mask=wr)
    plsc.store_scatter(pmax_v, [idx], pos, mask=wr)
    return jnp.logical_and(jnp.logical_and(act, jnp.logical_not(win)), bet)


def _body(msg_hbm, idx_hbm, t_hbm, ninf_hbm, neg1_hbm, out_hbm,
          idx_v, t_v, tseg_v, pmax_v, tmp_v, red_v,
          red_p, gs_p, amax_v, safe_v, rows_v, sem,
          sp_t, sp_p, sp_a):
    c = lax.axis_index("c")
    s = lax.axis_index("s")
    base_in = s * EPT
    lane = lax.iota(jnp.int32, L)

    pltpu.sync_copy(idx_hbm.at[pl.ds(base_in, EPT)], idx_v)
    pltpu.sync_copy(t_hbm.at[pl.ds(base_in, EPT)], t_v)
    pltpu.sync_copy(ninf_hbm, tseg_v)
    pltpu.sync_copy(neg1_hbm, pmax_v)
    pltpu.sync_copy(neg1_hbm, tmp_v)

    all_act = lane >= 0

    def _slow_fix(idx, tv, pos, rem):
        def cond(m):
            return _any_lane(m > 0)

        def body(m):
            return _lex_update(
                tseg_v, pmax_v, tmp_v, idx, tv, pos, m > 0
            ).astype(jnp.int32)

        lax.while_loop(cond, body, rem.astype(jnp.int32))

    def group(i, _):
        base = i * (U * L)
        idxs, tvs, poss, ws = [], [], [], []
        for j in range(U):
            idx = idx_v[pl.ds(base + j * L, L)]
            tv = t_v[pl.ds(base + j * L, L)]
            pos = base_in + base + j * L + lane
            plsc.store_scatter(tmp_v, [idx], pos)
            idxs.append(idx)
            tvs.append(tv)
            poss.append(pos)
        for j in range(U):
            ws.append(plsc.load_gather(tmp_v, [idxs[j]]))
        rems = []
        for j in range(U):
            win = ws[j] == poss[j]
            ct = plsc.load_gather(tseg_v, [idxs[j]])
            cp = plsc.load_gather(pmax_v, [idxs[j]])
            bet = jnp.logical_or(
                tvs[j] > ct,
                jnp.logical_and(tvs[j] == ct, poss[j] > cp))
            wr = jnp.logical_and(win, bet)
            plsc.store_scatter(tseg_v, [idxs[j]], tvs[j], mask=wr)
            plsc.store_scatter(pmax_v, [idxs[j]], poss[j], mask=wr)
            rems.append(jnp.logical_and(jnp.logical_not(win), bet))
        any_rem = rems[0]
        for j in range(1, U):
            any_rem = jnp.logical_or(any_rem, rems[j])

        @pl.when(_any_lane(any_rem))
        def _slow():
            for j in range(U):
                _slow_fix(idxs[j], tvs[j], poss[j], rems[j])

        return _

    lax.fori_loop(0, CHUNKS // U, group, None)

    for i in range(U * (CHUNKS // U), CHUNKS):
        idx = idx_v[pl.ds(i * L, L)]
        tv = t_v[pl.ds(i * L, L)]
        pos = base_in + i * L + lane
        rem = _lex_update(tseg_v, pmax_v, tmp_v, idx, tv, pos, all_act)

        @pl.when(_any_lane(rem))
        def _slow_tail():
            _slow_fix(idx, tv, pos, rem)

    pltpu.sync_copy(tseg_v, sp_t.at[s])
    pltpu.sync_copy(pmax_v, sp_p.at[s])
    plsc.subcore_barrier()
    pltpu.sync_copy(sp_t.at[:, pl.ds(s * SLICE, SLICE)], red_v)
    pltpu.sync_copy(sp_p.at[:, pl.ds(s * SLICE, SLICE)], red_p)

    def red(j, _):
        ta = red_v[0, pl.ds(j * L, L)]
        pa = red_p[0, pl.ds(j * L, L)]
        for k in range(1, NS):
            tk = red_v[k, pl.ds(j * L, L)]
            pk = red_p[k, pl.ds(j * L, L)]
            b = jnp.logical_or(tk > ta,
                               jnp.logical_and(tk == ta, pk > pa))
            ta = jnp.where(b, tk, ta)
            pa = jnp.where(b, pk, pa)
        gs_p[pl.ds(j * L, L)] = pa
        return _

    lax.fori_loop(0, SLICE // L, red, None)
    pltpu.sync_copy(gs_p, sp_a.at[pl.ds(s * SLICE, SLICE)])
    plsc.subcore_barrier()

    obase = c * (NS * OUT_PER) + s * OUT_PER
    pltpu.sync_copy(sp_a.at[pl.ds(obase, OUT_PER)], amax_v)
    for j in range(OUT_PER // L):
        a = amax_v[pl.ds(j * L, L)]
        safe_v[j // (GCH // L), pl.ds((j % (GCH // L)) * L, L)] = (
            jnp.maximum(a, 0))

    for cc in range(NGCH):
        @pl.when(obase + cc * GCH + GCH <= DIM)
        def _chunk():
            pltpu.async_copy(msg_hbm.at[safe_v.at[cc]], rows_v, sem).wait()

            def fix(g, _):
                a16 = amax_v[pl.ds(cc * GCH + g * L, L)]
                any_invalid = jnp.min(a16) < 0

                @pl.when(any_invalid)
                def _zero():
                    for r in range(L):
                        a_r = jnp.sum(jnp.where(lane == r, a16, 0))
                        m = jnp.where(a_r < 0, jnp.float32(0), jnp.float32(1))
                        row = g * L + r
                        for k in range(D // L):
                            rows_v[row, pl.ds(k * L, L)] = (
                                rows_v[row, pl.ds(k * L, L)] * m)

                return _

            lax.fori_loop(0, GCH // L, fix, None)
            pltpu.sync_copy(rows_v, out_hbm.at[pl.ds(obase + cc * GCH, GCH)])


@functools.partial(jax.jit, static_argnums=())
def kernel(msg, index, t, dim_size):
    del dim_size
    ninf = jnp.full((DIMP,), _NEG_INF, dtype=jnp.float32)
    neg1 = jnp.full((DIMP,), -1, dtype=jnp.int32)

    mesh = plsc.VectorSubcoreMesh(
        core_axis_name="c", subcore_axis_name="s",
        num_cores=NC, num_subcores=NS)
    run = pl.kernel(
        _body,
        out_type=jax.ShapeDtypeStruct((DIM, D), jnp.float32),
        mesh=mesh,
        compiler_params=pltpu.CompilerParams(needs_layout_passes=False),
        scratch_types=[
            pltpu.VMEM((EPT,), jnp.int32),
            pltpu.VMEM((EPT,), jnp.float32),
            pltpu.VMEM((DIMP,), jnp.float32),
            pltpu.VMEM((DIMP,), jnp.int32),
            pltpu.VMEM((DIMP,), jnp.int32),
            pltpu.VMEM((NS, SLICE), jnp.float32),
            pltpu.VMEM((NS, SLICE), jnp.int32),
            pltpu.VMEM((SLICE,), jnp.int32),
            pltpu.VMEM((OUT_PER,), jnp.int32),
            pltpu.VMEM((NGCH, GCH), jnp.int32),
            pltpu.VMEM((GCH, D), jnp.float32),
            pltpu.SemaphoreType.DMA,
            pltpu.VMEM_SHARED((NS, DIMP), jnp.float32),
            pltpu.VMEM_SHARED((NS, DIMP), jnp.int32),
            pltpu.VMEM_SHARED((DIMP,), jnp.int32),
        ],
    )
    return run(msg, index, t, ninf, neg1)


# speedup vs baseline: 19.1307x; 1.0022x over previous
import functools

import jax
import jax.numpy as jnp
from jax import lax
from jax.experimental import pallas as pl
from jax.experimental.pallas import tpu as pltpu
from jax.experimental.pallas import tpu_sc as plsc

N = 160000
D = 256
DIM = 10000
DIMP = 10240
NC = 2
NS = 16
L = 16
EPT = N // NS
CHUNKS = EPT // L
SLICE = DIMP // NS
OUT_PER = DIMP // (NC * NS)
GCH = 80
NGCH = OUT_PER // GCH
U = 4

_NEG_INF = float(jnp.finfo(jnp.float32).min)


def _any_lane(mask):
    cnt = plsc.all_reduce_population_count(mask)
    return jnp.squeeze(lax.slice(cnt, (0,), (1,))) > 0


def _lex_update(tseg_v, pmax_v, tmp_v, idx, tv, pos, act):
    plsc.store_scatter(tmp_v, [idx], pos, mask=act)
    w = plsc.load_gather(tmp_v, [idx])
    win = jnp.logical_and(act, w == pos)
    ct = plsc.load_gather(tseg_v, [idx])
    cp = plsc.load_gather(pmax_v, [idx])
    bet = jnp.logical_or(tv > ct,
                         jnp.logical_and(tv == ct, pos > cp))
    wr = jnp.logical_and(win, bet)
    plsc.store_scatter(tseg_v, [idx], tv, mask=wr)
    plsc.store_scatter(pmax_v, [idx], pos, mask=wr)
    return jnp.logical_and(jnp.logical_and(act, jnp.logical_not(win)), bet)


def _body(msg_hbm, idx_hbm, t_hbm, ninf_hbm, neg1_hbm, out_hbm,
          idx_v, t_v, tseg_v, pmax_v, tmp_v, red_v,
          red_p, gs_p, amax_v, safe_v, rows_v, sem,
          sp_t, sp_p, sp_a):
    c = lax.axis_index("c")
    s = lax.axis_index("s")
    base_in = s * EPT
    lane = lax.iota(jnp.int32, L)

    with jax.named_scope("stage_in"):
        pltpu.sync_copy(idx_hbm.at[pl.ds(base_in, EPT)], idx_v)
        pltpu.sync_copy(t_hbm.at[pl.ds(base_in, EPT)], t_v)
        pltpu.sync_copy(ninf_hbm, tseg_v)
        pltpu.sync_copy(neg1_hbm, pmax_v)
        pltpu.sync_copy(neg1_hbm, tmp_v)

    all_act = lane >= 0

    def _slow_fix(idx, tv, pos, rem):
        def cond(m):
            return _any_lane(m > 0)

        def body(m):
            return _lex_update(
                tseg_v, pmax_v, tmp_v, idx, tv, pos, m > 0
            ).astype(jnp.int32)

        lax.while_loop(cond, body, rem.astype(jnp.int32))

    def group(i, _):
        base = i * (U * L)
        idxs, tvs, poss, ws = [], [], [], []
        for j in range(U):
            idx = idx_v[pl.ds(base + j * L, L)]
            tv = t_v[pl.ds(base + j * L, L)]
            pos = base_in + base + j * L + lane
            plsc.store_scatter(tmp_v, [idx], pos)
            idxs.append(idx)
            tvs.append(tv)
            poss.append(pos)
        for j in range(U):
            ws.append(plsc.load_gather(tmp_v, [idxs[j]]))
        rems = []
        for j in range(U):
            win = ws[j] == poss[j]
            ct = plsc.load_gather(tseg_v, [idxs[j]])
            cp = plsc.load_gather(pmax_v, [idxs[j]])
            bet = jnp.logical_or(
                tvs[j] > ct,
                jnp.logical_and(tvs[j] == ct, poss[j] > cp))
            wr = jnp.logical_and(win, bet)
            plsc.store_scatter(tseg_v, [idxs[j]], tvs[j], mask=wr)
            plsc.store_scatter(pmax_v, [idxs[j]], poss[j], mask=wr)
            rems.append(jnp.logical_and(jnp.logical_not(win), bet))
        any_rem = rems[0]
        for j in range(1, U):
            any_rem = jnp.logical_or(any_rem, rems[j])

        @pl.when(_any_lane(any_rem))
        def _slow():
            for j in range(U):
                _slow_fix(idxs[j], tvs[j], poss[j], rems[j])

        return _

    with jax.named_scope("walk"):
        lax.fori_loop(0, CHUNKS // U, group, None)

    for i in range(U * (CHUNKS // U), CHUNKS):
        idx = idx_v[pl.ds(i * L, L)]
        tv = t_v[pl.ds(i * L, L)]
        pos = base_in + i * L + lane
        rem = _lex_update(tseg_v, pmax_v, tmp_v, idx, tv, pos, all_act)

        @pl.when(_any_lane(rem))
        def _slow_tail():
            _slow_fix(idx, tv, pos, rem)

    with jax.named_scope("publish"):
        pltpu.sync_copy(tseg_v, sp_t.at[s])
        pltpu.sync_copy(pmax_v, sp_p.at[s])
        plsc.subcore_barrier()
    with jax.named_scope("redcopy"):
        pltpu.sync_copy(sp_t.at[:, pl.ds(s * SLICE, SLICE)], red_v)
        pltpu.sync_copy(sp_p.at[:, pl.ds(s * SLICE, SLICE)], red_p)

    def red(j, _):
        ta = red_v[0, pl.ds(j * L, L)]
        pa = red_p[0, pl.ds(j * L, L)]
        for k in range(1, NS):
            tk = red_v[k, pl.ds(j * L, L)]
            pk = red_p[k, pl.ds(j * L, L)]
            b = jnp.logical_or(tk > ta,
                               jnp.logical_and(tk == ta, pk > pa))
            ta = jnp.where(b, tk, ta)
            pa = jnp.where(b, pk, pa)
        gs_p[pl.ds(j * L, L)] = pa
        return _

    with jax.named_scope("reduce"):
        lax.fori_loop(0, SLICE // L, red, None)
        pltpu.sync_copy(gs_p, sp_a.at[pl.ds(s * SLICE, SLICE)])
        plsc.subcore_barrier()

    obase = c * (NS * OUT_PER) + s * OUT_PER
    scope3 = jax.named_scope("rowgather")
    scope3.__enter__()
    pltpu.sync_copy(sp_a.at[pl.ds(obase, OUT_PER)], amax_v)
    for j in range(OUT_PER // L):
        a = amax_v[pl.ds(j * L, L)]
        safe_v[j // (GCH // L), pl.ds((j % (GCH // L)) * L, L)] = (
            jnp.maximum(a, 0))

    for cc in range(NGCH):
        @pl.when(obase + cc * GCH + GCH <= DIM)
        def _chunk():
            pltpu.async_copy(msg_hbm.at[safe_v.at[cc]], rows_v, sem).wait()

            def fix(g, _):
                a16 = amax_v[pl.ds(cc * GCH + g * L, L)]
                any_invalid = jnp.min(a16) < 0

                @pl.when(any_invalid)
                def _zero():
                    for r in range(L):
                        a_r = jnp.sum(jnp.where(lane == r, a16, 0))
                        m = jnp.where(a_r < 0, jnp.float32(0), jnp.float32(1))
                        row = g * L + r
                        for k in range(D // L):
                            rows_v[row, pl.ds(k * L, L)] = (
                                rows_v[row, pl.ds(k * L, L)] * m)

                return _

            lax.fori_loop(0, GCH // L, fix, None)
            pltpu.sync_copy(rows_v, out_hbm.at[pl.ds(obase + cc * GCH, GCH)])

    scope3.__exit__(None, None, None)


@functools.partial(jax.jit, static_argnums=())
def kernel(msg, index, t, dim_size):
    del dim_size
    ninf = jnp.full((DIMP,), _NEG_INF, dtype=jnp.float32)
    neg1 = jnp.full((DIMP,), -1, dtype=jnp.int32)

    mesh = plsc.VectorSubcoreMesh(
        core_axis_name="c", subcore_axis_name="s",
        num_cores=NC, num_subcores=NS)
    run = pl.kernel(
        _body,
        out_type=jax.ShapeDtypeStruct((DIM, D), jnp.float32),
        mesh=mesh,
        compiler_params=pltpu.CompilerParams(needs_layout_passes=False),
        scratch_types=[
            pltpu.VMEM((EPT,), jnp.int32),
            pltpu.VMEM((EPT,), jnp.float32),
            pltpu.VMEM((DIMP,), jnp.float32),
            pltpu.VMEM((DIMP,), jnp.int32),
            pltpu.VMEM((DIMP,), jnp.int32),
            pltpu.VMEM((NS, SLICE), jnp.float32),
            pltpu.VMEM((NS, SLICE), jnp.int32),
            pltpu.VMEM((SLICE,), jnp.int32),
            pltpu.VMEM((OUT_PER,), jnp.int32),
            pltpu.VMEM((NGCH, GCH), jnp.int32),
            pltpu.VMEM((GCH, D), jnp.float32),
            pltpu.SemaphoreType.DMA,
            pltpu.VMEM_SHARED((NS, DIMP), jnp.float32),
            pltpu.VMEM_SHARED((NS, DIMP), jnp.int32),
            pltpu.VMEM_SHARED((DIMP,), jnp.int32),
        ],
    )
    return run(msg, index, t, ninf, neg1)
